# baseline (speedup 1.0000x reference)
import functools

import jax
import jax.numpy as jnp
from jax import lax
from jax.experimental import pallas as pl
from jax.experimental.pallas import tpu as pltpu
from jax.experimental.pallas import tpu_sc as plsc

N = 10000
E = 80000
F = 32
D = 16

NC, NS = 2, 16
NW = NC * NS
EPW = 2560
NCH = 20
NP = 10240
RPT = NP // NS
EQ = E // 4

_MESH = dict(core_axis_name="c", subcore_axis_name="s")
_SC_PARAMS = pltpu.CompilerParams(use_tc_tiling_on_sc=False)


def _sc_gather(table, idx3):
    @functools.partial(
        pl.kernel,
        out_type=jax.ShapeDtypeStruct((E, F), jnp.float32),
        mesh=plsc.VectorSubcoreMesh(**_MESH),
        scratch_types=[
            pltpu.VMEM((NCH, 128), jnp.int32),
            pltpu.VMEM((EPW, F), jnp.float32),
            pltpu.SemaphoreType.DMA,
        ],
        compiler_params=_SC_PARAMS,
    )
    def k(table_hbm, idx_hbm, out_hbm, idx_v, rows_v, sem):
        cid = lax.axis_index("c")
        sid = lax.axis_index("s")
        wid = sid * NC + cid
        pltpu.sync_copy(idx_hbm.at[wid], idx_v)
        head = [
            pltpu.async_copy(table_hbm.at[idx_v.at[j]],
                             rows_v.at[pl.ds(j * 128, 128)], sem)
            for j in range(5)
        ]

        @pl.when(wid < NW - 1)
        def _tail_gather():
            tail = [
                pltpu.async_copy(table_hbm.at[idx_v.at[j]],
                                 rows_v.at[pl.ds(j * 128, 128)], sem)
                for j in range(5, NCH)
            ]
            for c in tail:
                c.wait()

        for c in head:
            c.wait()
        pltpu.sync_copy(rows_v.at[pl.ds(0, 640)],
                        out_hbm.at[pl.ds(wid * EPW, 640)])

        @pl.when(wid < NW - 1)
        def _tail_out():
            pltpu.sync_copy(rows_v.at[pl.ds(640, 1920)],
                            out_hbm.at[pl.ds(wid * EPW + 640, 1920)])

    return k(table, idx3)


def _sc_pack_e(eT):
    @functools.partial(
        pl.kernel,
        out_type=jax.ShapeDtypeStruct((EQ * 128,), jnp.float32),
        mesh=plsc.VectorSubcoreMesh(**_MESH),
        scratch_types=[
            pltpu.VMEM((EPW * D,), jnp.float32),
            pltpu.VMEM((EPW * 32,), jnp.float32),
        ],
        compiler_params=_SC_PARAMS,
    )
    def k(ef_hbm, out_hbm, et_v, o_v):
        cid = lax.axis_index("c")
        sid = lax.axis_index("s")
        wid = sid * NC + cid
        pltpu.sync_copy(ef_hbm.at[pl.ds(wid * EPW * D, 640 * D)],
                        et_v.at[pl.ds(0, 640 * D)])

        @pl.when(wid < NW - 1)
        def _tail_in():
            pltpu.sync_copy(ef_hbm.at[pl.ds(wid * EPW * D + 640 * D,
                                            1920 * D)],
                            et_v.at[pl.ds(640 * D, 1920 * D)])

        zeros16 = jnp.zeros((16,), jnp.float32)
        nq = jnp.where(wid < NW - 1, EPW // 4, 160)

        def row(q, carry):
            for kk in range(4):
                vec = et_v[pl.ds((4 * q + kk) * D, 16)]
                o_v[pl.ds(q * 128 + kk * 16, 16)] = vec
                o_v[pl.ds(q * 128 + 64 + kk * 16, 16)] = zeros16
            return carry

        lax.fori_loop(0, nq, row, 0)
        pltpu.sync_copy(o_v.at[pl.ds(0, 160 * 128)],
                        out_hbm.at[pl.ds(wid * (EPW * 32), 160 * 128)])

        @pl.when(wid < NW - 1)
        def _tail_out():
            pltpu.sync_copy(
                o_v.at[pl.ds(160 * 128, 480 * 128)],
                out_hbm.at[pl.ds(wid * (EPW * 32) + 160 * 128, 480 * 128)])

    return k(eT)


def _sc_scatter(m, tgt3, zeros_np):
    @functools.partial(
        pl.kernel,
        out_type=jax.ShapeDtypeStruct((NC, NP, F), jnp.float32),
        mesh=plsc.VectorSubcoreMesh(**_MESH),
        scratch_types=[
            pltpu.VMEM((NCH, 128), jnp.int32),
            pltpu.VMEM((EPW, F), jnp.float32),
            pltpu.VMEM((RPT, F), jnp.float32),
            pltpu.VMEM_SHARED((NP, F), jnp.float32),
            pltpu.SemaphoreType.DMA,
        ],
        compiler_params=_SC_PARAMS,
    )
    def k(m_hbm, tgt_hbm, z_hbm, p_hbm, idx_v, m_v, stage_v, acc_sh, sem):
        cid = lax.axis_index("c")
        sid = lax.axis_index("s")
        wid = sid * NC + cid
        pltpu.sync_copy(z_hbm.at[pl.ds(sid * RPT, RPT)], stage_v)
        pltpu.sync_copy(stage_v, acc_sh.at[pl.ds(sid * RPT, RPT)])
        pltpu.sync_copy(tgt_hbm.at[wid], idx_v)
        pltpu.sync_copy(m_hbm.at[pl.ds(wid * EPW, 640)],
                        m_v.at[pl.ds(0, 640)])

        @pl.when(wid < NW - 1)
        def _tail_in():
            pltpu.sync_copy(m_hbm.at[pl.ds(wid * EPW + 640, 1920)],
                            m_v.at[pl.ds(640, 1920)])

        plsc.subcore_barrier()
        head = [
            pltpu.async_copy(m_v.at[pl.ds(j * 128, 128)],
                             acc_sh.at[idx_v.at[j]], sem, add=True)
            for j in range(5)
        ]

        @pl.when(wid < NW - 1)
        def _tail_add():
            tail = [
                pltpu.async_copy(m_v.at[pl.ds(j * 128, 128)],
                                 acc_sh.at[idx_v.at[j]], sem, add=True)
                for j in range(5, NCH)
            ]
            for c in tail:
                c.wait()

        for c in head:
            c.wait()
        plsc.subcore_barrier()
        pltpu.sync_copy(acc_sh.at[pl.ds(sid * RPT, RPT)], stage_v)
        pltpu.sync_copy(stage_v, p_hbm.at[cid, pl.ds(sid * RPT, RPT)])

    return k(m, tgt3, zeros_np)


def _tc_messages(e4, msgs_p, W4, B4, xh, root, bias1r):
    GRID = 25
    TQ = EQ // GRID
    NB = N // GRID

    v = jnp.arange(4 * D * F)
    jj = v // (D * F)
    dd = (v % (D * F)) // F
    bb = v % F
    S4 = jnp.concatenate([
        (jnp.arange(4 * D)[:, None] == (jj * D + dd)[None, :]
         ).astype(jnp.bfloat16),
        jnp.zeros((128 - 4 * D, 4 * D * F), jnp.bfloat16),
    ])
    T4 = (jnp.arange(4 * F)[:, None] == (jj * F + bb)[None, :]
          ).astype(jnp.bfloat16)
    dn = (((1,), (0,)), ((), ()))

    def body(e_ref, mg_ref, s_ref, t_ref, w4_ref, b4_ref, x_ref, root_ref,
             b_ref, m_ref, r_ref):
        eb = e_ref[...].astype(jnp.bfloat16)
        mp = mg_ref[...]
        mp16 = mp.astype(jnp.bfloat16)
        e_rep = jax.lax.dot_general(eb, s_ref[...], dn,
                                    preferred_element_type=jnp.float32)
        m_rep = jax.lax.dot_general(mp16, t_ref[...], dn,
                                    preferred_element_type=jnp.float32)
        z = (e_rep * m_rep).astype(jnp.bfloat16)
        m = jax.lax.dot_general(z, w4_ref[...], dn,
                                preferred_element_type=jnp.float32)
        m_ref[...] = m + jax.lax.dot_general(
            mp, b4_ref[...], dn, preferred_element_type=jnp.float32)
        r_ref[...] = x_ref[...] @ root_ref[...] + b_ref[...]

    return pl.pallas_call(
        body,
        grid=(GRID,),
        in_specs=[
            pl.BlockSpec((TQ, 128), lambda i: (i, 0)),
            pl.BlockSpec((TQ, 4 * F), lambda i: (i, 0)),
            pl.BlockSpec((128, 4 * D * F), lambda i: (0, 0)),
            pl.BlockSpec((4 * F, 4 * D * F), lambda i: (0, 0)),
            pl.BlockSpec((4 * D * F, 4 * F), lambda i: (0, 0)),
            pl.BlockSpec((4 * F, 4 * F), lambda i: (0, 0)),
            pl.BlockSpec((NB, F), lambda i: (i, 0)),
            pl.BlockSpec((F, F), lambda i: (0, 0)),
            pl.BlockSpec((1, F), lambda i: (0, 0)),
        ],
        out_specs=[
            pl.BlockSpec((TQ, 4 * F), lambda i: (i, 0)),
            pl.BlockSpec((NB, F), lambda i: (i, 0)),
        ],
        out_shape=[
            jax.ShapeDtypeStruct((EQ, 4 * F), jnp.float32),
            jax.ShapeDtypeStruct((N, F), jnp.float32),
        ],
    )(e4, msgs_p, S4, T4, W4, B4, xh, root, bias1r)


def _tc_relu3(pa, pb, r):
    def body(a_ref, b_ref, r_ref, h_ref):
        h_ref[...] = jnp.maximum(a_ref[...] + b_ref[...] + r_ref[...], 0.0)

    return pl.pallas_call(
        body,
        out_shape=jax.ShapeDtypeStruct((N, F), jnp.float32),
    )(pa, pb, r)


def _tc_final(pa, pb, r, dw, db):
    def body(a_ref, b_ref, r_ref, w_ref, db_ref, o_ref):
        h = jnp.maximum(a_ref[...] + b_ref[...] + r_ref[...], 0.0)
        pooled = jnp.sum(h, axis=0, keepdims=True)
        o_ref[...] = pooled @ w_ref[...] + db_ref[...]

    return pl.pallas_call(
        body,
        out_shape=jax.ShapeDtypeStruct((1, 1), jnp.float32),
    )(pa, pb, r, dw, db.reshape(1, 1))


def _expand_w(w_k, b_k):
    Wt = w_k.reshape(D * F, F).astype(jnp.bfloat16)
    Bm = b_k.reshape(F, F)
    eye4b = jnp.eye(4, dtype=jnp.bfloat16)
    eye4f = jnp.eye(4, dtype=jnp.float32)
    W4 = (eye4b[:, None, :, None] * Wt[None, :, None, :]
          ).reshape(4 * D * F, 4 * F)
    B4 = (eye4f[:, None, :, None] * Bm[None, :, None, :]
          ).reshape(4 * F, 4 * F)
    return W4, B4


def kernel(x, edge_index, e, w_k1, b_k1, root1, bias1,
           w_k2, b_k2, root2, bias2, dense_w, dense_b):
    src = edge_index[0]
    tgt = edge_index[1]
    pad = jnp.zeros((NW * NCH * 128 - E,), jnp.int32)
    src3 = jnp.concatenate([src, pad]).reshape(NW, NCH, 128)
    tgt3 = jnp.concatenate([tgt, pad]).reshape(NW, NCH, 128)
    e4 = _sc_pack_e(e.reshape(E * D)).reshape(EQ, 128)
    W41, B41 = _expand_w(w_k1, b_k1)
    W42, B42 = _expand_w(w_k2, b_k2)
    zeros_np = jnp.zeros((NP, F), jnp.float32)

    msgs1 = _sc_gather(x, src3)
    m1, r1 = _tc_messages(e4, msgs1.reshape(EQ, 4 * F), W41, B41, x,
                          root1, bias1.reshape(1, F))
    p1 = _sc_scatter(m1.reshape(E, F), tgt3, zeros_np)
    h1 = _tc_relu3(p1[0, :N], p1[1, :N], r1)
    msgs2 = _sc_gather(h1, src3)
    m2, r2 = _tc_messages(e4, msgs2.reshape(EQ, 4 * F), W42, B42, h1,
                          root2, bias2.reshape(1, F))
    p2 = _sc_scatter(m2.reshape(E, F), tgt3, zeros_np)
    return _tc_final(p2[0, :N], p2[1, :N], r2[:N], dense_w, dense_b)
